# speedup vs baseline: 1.5010x; 1.5010x over previous
import functools

import jax
import jax.numpy as jnp
from jax import lax
from jax.experimental import pallas as pl
from jax.experimental.pallas import tpu as pltpu
from jax.experimental.pallas import tpu_sc as plsc

_NC, _NS, _L = 2, 16, 16
_NW = _NC * _NS
_D = 256
_B = 64 * 1024
_ROWS = 2 * _B
_RPW = _ROWS // _NW
_CH = 128
_NCH = _RPW // _CH
_IDXROWS = _RPW // _CH


@functools.partial(
    pl.kernel,
    mesh=plsc.VectorSubcoreMesh(core_axis_name="c", subcore_axis_name="s"),
    out_type=jax.ShapeDtypeStruct((_ROWS, _D), jnp.float32),
    scratch_types=[
        pltpu.VMEM((_IDXROWS, _CH), jnp.int32),
        pltpu.VMEM((_CH, _D), jnp.float32),
        pltpu.SemaphoreType.DMA,
    ],
)
def _sc_lookup(idx_hbm, table_hbm, out_hbm, idx_v, buf, sg):
    wid = lax.axis_index("s") * _NC + lax.axis_index("c")
    base = wid * _RPW

    pltpu.sync_copy(idx_hbm.at[pl.ds(wid * _IDXROWS, _IDXROWS)], idx_v)

    off = (lax.iota(jnp.int32, _L) % 2) * 64

    def add_row(i, carry):
        def add_vec(j, c2):
            sl = pl.ds(j * _L, _L)
            idx_v[i, sl] = idx_v[i, sl] + off
            return c2
        return lax.fori_loop(0, _CH // _L, add_vec, carry)

    lax.fori_loop(0, _IDXROWS, add_row, 0)

    def chunk(c, carry):
        pltpu.async_copy(table_hbm.at[idx_v.at[c]], buf, sg).wait()
        pltpu.sync_copy(buf, out_hbm.at[pl.ds(base + c * _CH, _CH)])
        return carry

    lax.fori_loop(0, _NCH, chunk, 0)


def kernel(position_inds, col_embed, row_embed):
    table = jnp.concatenate([col_embed, row_embed], axis=0)
    idx = position_inds.astype(jnp.int32).reshape(_ROWS // _CH, _CH)
    out = _sc_lookup(idx, table)
    return out.reshape(64, 1024, 2 * _D)
